# baseline (speedup 1.0000x reference)
import functools

import jax
import jax.numpy as jnp
from jax import lax
from jax.experimental import pallas as pl
from jax.experimental.pallas import tpu as pltpu
from jax.experimental.pallas import tpu_sc as plsc

USER_NUM = 20000
ITEM_NUM = 80000
N = USER_NUM + ITEM_NUM
NPAD = 100352
E = 1600000
D = 64
NGRP = 4

T_EDGES = 100352
EPAD = 16 * T_EDGES
QE = 512
NQROW = T_EDGES // QE
NSUP = NQROW // 4
ROWS_PER_TILE = NPAD // 16
ZB = 98
WB = 98

T2_EDGES = EPAD // 32
NB2 = T2_EDGES // 128
SB2 = 56
NSUP2 = NB2 // SB2


@functools.cache
def _mesh():
    return plsc.VectorSubcoreMesh(core_axis_name="c", subcore_axis_name="s",
                                  num_cores=2, num_subcores=16)


def _deg_body(dst2d, degp, deg_sh, dst_c, ones_v, zbuf1):
    c = lax.axis_index("c")
    s = lax.axis_index("s")
    wid = c * 16 + s
    zero16 = jnp.zeros((16,), jnp.float32)
    one16 = jnp.ones((16,), jnp.float32)

    def _z(i, carry):
        zbuf1[pl.ds(i * 16, 16)] = zero16
        return carry

    lax.fori_loop(0, ROWS_PER_TILE // 16, _z, 0)
    for k in range(8):
        ones_v[pl.ds(k * 16, 16)] = one16
    pltpu.sync_copy(zbuf1, deg_sh.at[pl.ds(s * ROWS_PER_TILE, ROWS_PER_TILE)])
    plsc.subcore_barrier()

    for u in range(NSUP2):
        row0 = wid * NB2 + u * SB2
        pltpu.sync_copy(dst2d.at[pl.ds(row0, SB2)], dst_c)

        def _b(b, carry):
            pltpu.sync_copy(ones_v, deg_sh.at[dst_c.at[b]], add=True)
            return carry

        lax.fori_loop(0, SB2, _b, 0)

    plsc.subcore_barrier()
    pltpu.sync_copy(deg_sh.at[pl.ds(s * ROWS_PER_TILE, ROWS_PER_TILE)],
                    degp.at[c, pl.ds(s * ROWS_PER_TILE, ROWS_PER_TILE)])


@functools.cache
def _deg_call():
    return pl.kernel(
        _deg_body,
        out_type=jax.ShapeDtypeStruct((2, NPAD), jnp.float32),
        mesh=_mesh(),
        compiler_params=pltpu.CompilerParams(use_tc_tiling_on_sc=False),
        scratch_types=[
            pltpu.VMEM_SHARED((NPAD + 8,), jnp.float32),
            pltpu.VMEM((SB2, 128), jnp.int32),
            pltpu.VMEM((128,), jnp.float32),
            pltpu.VMEM((ROWS_PER_TILE,), jnp.float32),
        ],
    )


def _mega_body(z0, src2d, dst2d, dsq, s1, s2, s3, z1, z2,
               accum, src_c, dst_c, rows0, rows1, rows2, rows3,
               zbuf, wb_buf, dsq_buf, zwb_buf, semA, semB, semC, semD):
    c = lax.axis_index("c")
    s = lax.axis_index("s")
    zero16 = jnp.zeros((16,), jnp.float32)

    def _zrow(i, carry):
        zbuf[i, :] = zero16
        return carry

    lax.fori_loop(0, ZB, _zrow, 0)

    rows = (rows0, rows1, rows2, rows3)
    sems = (semA, semB, semC, semD)
    zs_in = (z0, z1, z2)
    zs_out = (z1, z2, None)
    ss = (s1, s2, s3)

    for layer in range(3):
        zf = zs_in[layer]
        for p in range(2):
            g = c * 2 + p

            def _zcopy(k, carry):
                pltpu.sync_copy(
                    zbuf, accum.at[pl.ds(s * ROWS_PER_TILE + k * ZB, ZB)])
                return carry

            lax.fori_loop(0, ROWS_PER_TILE // ZB, _zcopy, 0)
            plsc.subcore_barrier()

            def _superchunk(u, carry):
                row0 = s * NQROW + u * 4
                pltpu.sync_copy(src2d.at[pl.ds(row0, 4)], src_c)
                pltpu.sync_copy(dst2d.at[pl.ds(row0, 4)], dst_c)

                def _prep(i, carry2):
                    jq = i // 32
                    kk = i % 32
                    v = src_c[jq, pl.ds(kk * 16, 16)]
                    src_c[jq, pl.ds(kk * 16, 16)] = v + g * NPAD
                    return carry2

                lax.fori_loop(0, 4 * 32, _prep, 0)

                def _idx(ref, t):
                    return ref.at[t // 2, pl.ds((t % 2) * 256, 256)]

                def _fire(t):
                    par = t % 4
                    pltpu.async_copy(zf.at[_idx(src_c, t)], rows[par],
                                     sems[par])

                def _wait(t):
                    par = t % 4
                    pltpu.make_async_copy(zf.at[_idx(src_c, t)], rows[par],
                                          sems[par]).wait()

                def _scat(t):
                    par = t % 4
                    pltpu.sync_copy(rows[par], accum.at[_idx(dst_c, t)],
                                    add=True)

                for t in range(4):
                    _fire(t)
                for t in range(8):
                    _wait(t)
                    if t + 4 < 8:
                        _fire(t + 4)
                    _scat(t)
                return carry

            lax.fori_loop(0, NSUP, _superchunk, 0)
            plsc.subcore_barrier()

            def _wb(k, carry):
                r0 = s * ROWS_PER_TILE + k * WB
                pltpu.sync_copy(accum.at[pl.ds(r0, WB)], wb_buf)
                pltpu.sync_copy(
                    wb_buf, ss[layer].at[pl.ds(r0, WB), pl.ds(g * 16, 16)])
                if layer < 2:
                    pltpu.sync_copy(dsq.at[pl.ds(r0, WB)], dsq_buf)

                    def _mul(r, carry2):
                        zwb_buf[r, :] = wb_buf[r, :] * dsq_buf[r, :]
                        return carry2

                    lax.fori_loop(0, WB, _mul, 0)
                    pltpu.sync_copy(
                        zwb_buf,
                        zs_out[layer].at[pl.ds(g * NPAD + r0, WB)])
                return carry

            lax.fori_loop(0, ROWS_PER_TILE // WB, _wb, 0)
            plsc.subcore_barrier()


@functools.cache
def _mega_call():
    return pl.kernel(
        _mega_body,
        out_type=(
            jax.ShapeDtypeStruct((NPAD, D), jnp.float32),
            jax.ShapeDtypeStruct((NPAD, D), jnp.float32),
            jax.ShapeDtypeStruct((NPAD, D), jnp.float32),
            jax.ShapeDtypeStruct((NGRP * NPAD, 16), jnp.float32),
            jax.ShapeDtypeStruct((NGRP * NPAD, 16), jnp.float32),
        ),
        mesh=_mesh(),
        compiler_params=pltpu.CompilerParams(use_tc_tiling_on_sc=False),
        scratch_types=[
            pltpu.VMEM_SHARED((NPAD + 8, 16), jnp.float32),
            pltpu.VMEM((4, QE), jnp.int32),
            pltpu.VMEM((4, QE), jnp.int32),
            pltpu.VMEM((256, 16), jnp.float32),
            pltpu.VMEM((256, 16), jnp.float32),
            pltpu.VMEM((256, 16), jnp.float32),
            pltpu.VMEM((256, 16), jnp.float32),
            pltpu.VMEM((ZB, 16), jnp.float32),
            pltpu.VMEM((WB, 16), jnp.float32),
            pltpu.VMEM((WB, 16), jnp.float32),
            pltpu.VMEM((WB, 16), jnp.float32),
            pltpu.SemaphoreType.DMA,
            pltpu.SemaphoreType.DMA,
            pltpu.SemaphoreType.DMA,
            pltpu.SemaphoreType.DMA,
        ],
    )


_BLK = 6272
_GRID = NPAD // _BLK


def _scale_body(a_ref, b_ref, o_ref):
    o_ref[:, :] = a_ref[:, :] * b_ref[:, :]


_scale_call = pl.pallas_call(
    _scale_body,
    grid=(_GRID,),
    in_specs=[pl.BlockSpec((_BLK, D), lambda i: (i, 0)),
              pl.BlockSpec((_BLK, D), lambda i: (i, 0))],
    out_specs=pl.BlockSpec((_BLK, D), lambda i: (i, 0)),
    out_shape=jax.ShapeDtypeStruct((NPAD, D), jnp.float32),
)


def _dsq_body(d_ref, o_ref):
    d16 = d_ref[:, 0:16]
    o_ref[:, :] = d16 * d16


_dsq_call = pl.pallas_call(
    _dsq_body,
    grid=(_GRID,),
    in_specs=[pl.BlockSpec((_BLK, D), lambda j: (j, 0))],
    out_specs=pl.BlockSpec((_BLK, 16), lambda j: (j, 0)),
    out_shape=jax.ShapeDtypeStruct((NPAD, 16), jnp.float32),
)


def _final_body(x_ref, d_ref, s1_ref, s2_ref, s3_ref, o_ref):
    acc = s1_ref[:, :] + s2_ref[:, :] + s3_ref[:, :]
    o_ref[:, :] = (x_ref[:, :] + d_ref[:, :] * acc) * (1.0 / 16.0)


_final_call = pl.pallas_call(
    _final_body,
    grid=(_GRID,),
    in_specs=[pl.BlockSpec((_BLK, D), lambda i: (i, 0)) for _ in range(5)],
    out_specs=pl.BlockSpec((_BLK, D), lambda i: (i, 0)),
    out_shape=jax.ShapeDtypeStruct((NPAD, D), jnp.float32),
)


def kernel(edge_index, user_int, item_int, user_geo, item_geo):
    src = edge_index[0].astype(jnp.int32)
    dst = edge_index[1].astype(jnp.int32)

    x_int = jnp.concatenate([user_int, item_int], axis=0)
    x_geo = jnp.concatenate([user_geo, item_geo], axis=0)
    x0 = jnp.concatenate([x_int, x_geo], axis=1)
    x0p = jnp.pad(x0, ((0, NPAD - N), (0, 0)))

    pad_e = EPAD - E
    src_p = jnp.concatenate([src, jnp.zeros((pad_e,), jnp.int32)])
    dst_p = jnp.concatenate([dst, jnp.full((pad_e,), NPAD, jnp.int32)])
    src3d = src_p.reshape(EPAD // QE, QE)
    dst3d = dst_p.reshape(EPAD // QE, QE)
    dst2d = dst_p.reshape(EPAD // 128, 128)

    degp = _deg_call()(dst2d)
    deg = degp[0] + degp[1]
    dis = jnp.where(deg > 0, 1.0 / jnp.sqrt(jnp.where(deg > 0, deg, 1.0)),
                    0.0)
    disb = jnp.broadcast_to(dis[:, None], (NPAD, D))

    zfull = _scale_call(x0p, disb)
    z0 = zfull.reshape(NPAD, NGRP, 16).transpose(1, 0, 2).reshape(
        NGRP * NPAD, 16)
    dsq16 = _dsq_call(disb)
    s1, s2, s3, _z1, _z2 = _mega_call()(z0, src3d, dst3d, dsq16)

    out = _final_call(x0p, disb, s1, s2, s3)

    return (out[:USER_NUM, :32], out[USER_NUM:N, :32],
            out[:USER_NUM, 32:], out[USER_NUM:N, 32:])


# speedup vs baseline: 1.0828x; 1.0828x over previous
import functools

import jax
import jax.numpy as jnp
from jax import lax
from jax.experimental import pallas as pl
from jax.experimental.pallas import tpu as pltpu
from jax.experimental.pallas import tpu_sc as plsc

USER_NUM = 20000
ITEM_NUM = 80000
N = USER_NUM + ITEM_NUM
NPAD = 100352
E = 1600000
D = 64
NGRP = 4

T_EDGES = 100352
EPAD = 16 * T_EDGES
QE = 512
NQROW = T_EDGES // QE
NSUP = NQROW // 4
ROWS_PER_TILE = NPAD // 16
ZB = 98
WB = 196

T2_EDGES = EPAD // 32
NB2 = T2_EDGES // 128
SB2 = 56
NSUP2 = NB2 // SB2


@functools.cache
def _mesh():
    return plsc.VectorSubcoreMesh(core_axis_name="c", subcore_axis_name="s",
                                  num_cores=2, num_subcores=16)


def _deg_body(dst2d, degp, deg_sh, dst_c, ones_v, zbuf1):
    c = lax.axis_index("c")
    s = lax.axis_index("s")
    wid = c * 16 + s
    zero16 = jnp.zeros((16,), jnp.float32)
    one16 = jnp.ones((16,), jnp.float32)

    def _z(i, carry):
        zbuf1[pl.ds(i * 16, 16)] = zero16
        return carry

    lax.fori_loop(0, ROWS_PER_TILE // 16, _z, 0)
    for k in range(8):
        ones_v[pl.ds(k * 16, 16)] = one16
    pltpu.sync_copy(zbuf1, deg_sh.at[pl.ds(s * ROWS_PER_TILE, ROWS_PER_TILE)])
    plsc.subcore_barrier()

    for u in range(NSUP2):
        row0 = wid * NB2 + u * SB2
        pltpu.sync_copy(dst2d.at[pl.ds(row0, SB2)], dst_c)

        def _b(b, carry):
            pltpu.sync_copy(ones_v, deg_sh.at[dst_c.at[b]], add=True)
            return carry

        lax.fori_loop(0, SB2, _b, 0)

    plsc.subcore_barrier()
    pltpu.sync_copy(deg_sh.at[pl.ds(s * ROWS_PER_TILE, ROWS_PER_TILE)],
                    degp.at[c, pl.ds(s * ROWS_PER_TILE, ROWS_PER_TILE)])


@functools.cache
def _deg_call():
    return pl.kernel(
        _deg_body,
        out_type=jax.ShapeDtypeStruct((2, NPAD), jnp.float32),
        mesh=_mesh(),
        compiler_params=pltpu.CompilerParams(use_tc_tiling_on_sc=False),
        scratch_types=[
            pltpu.VMEM_SHARED((NPAD + 8,), jnp.float32),
            pltpu.VMEM((SB2, 128), jnp.int32),
            pltpu.VMEM((128,), jnp.float32),
            pltpu.VMEM((ROWS_PER_TILE,), jnp.float32),
        ],
    )


def _mega_body(z0, src2d, dst2d, dsq, s1, s2, s3, z1, z2,
               accum, src_c, dst_c, rows0, rows1, rows2, rows3,
               zbuf, wb_buf, dsq_buf, semA, semB, semC, semD):
    c = lax.axis_index("c")
    s = lax.axis_index("s")
    zero16 = jnp.zeros((16,), jnp.float32)

    def _zrow(i, carry):
        zbuf[i, :] = zero16
        return carry

    lax.fori_loop(0, ZB, _zrow, 0)

    rows = (rows0, rows1, rows2, rows3)
    sems = (semA, semB, semC, semD)
    zs_in = (z0, z1, z2)
    zs_out = (z1, z2, None)
    ss = (s1, s2, s3)

    for layer in range(3):
        zf = zs_in[layer]
        for p in range(2):
            g = c * 2 + p

            def _zcopy(k, carry):
                pltpu.sync_copy(
                    zbuf, accum.at[pl.ds(s * ROWS_PER_TILE + k * ZB, ZB)])
                return carry

            lax.fori_loop(0, ROWS_PER_TILE // ZB, _zcopy, 0)
            plsc.subcore_barrier()

            def _superchunk(u, carry):
                row0 = s * NQROW + u * 4
                pltpu.sync_copy(src2d.at[pl.ds(row0, 4)], src_c)
                pltpu.sync_copy(dst2d.at[pl.ds(row0, 4)], dst_c)

                def _prep(i, carry2):
                    jq = i // 32
                    kk = i % 32
                    v = src_c[jq, pl.ds(kk * 16, 16)]
                    if layer == 0:
                        src_c[jq, pl.ds(kk * 16, 16)] = v * NGRP + g
                    else:
                        src_c[jq, pl.ds(kk * 16, 16)] = v + g * NPAD
                    return carry2

                lax.fori_loop(0, 4 * 32, _prep, 0)

                def _idx(ref, t):
                    return ref.at[t // 2, pl.ds((t % 2) * 256, 256)]

                def _fire(t):
                    par = t % 4
                    pltpu.async_copy(zf.at[_idx(src_c, t)], rows[par],
                                     sems[par])

                def _wait(t):
                    par = t % 4
                    pltpu.make_async_copy(zf.at[_idx(src_c, t)], rows[par],
                                          sems[par]).wait()

                def _scat(t):
                    par = t % 4
                    pltpu.sync_copy(rows[par], accum.at[_idx(dst_c, t)],
                                    add=True)

                for t in range(4):
                    _fire(t)
                for t in range(8):
                    _wait(t)
                    if t + 4 < 8:
                        _fire(t + 4)
                    _scat(t)
                return carry

            lax.fori_loop(0, NSUP, _superchunk, 0)
            plsc.subcore_barrier()

            def _wb(k, carry):
                r0 = s * ROWS_PER_TILE + k * WB
                pltpu.sync_copy(accum.at[pl.ds(r0, WB)], wb_buf)
                pltpu.sync_copy(
                    wb_buf, ss[layer].at[pl.ds(r0, WB), pl.ds(g * 16, 16)])
                if layer < 2:
                    pltpu.sync_copy(dsq.at[pl.ds(r0, WB)], dsq_buf)

                    def _mul(r, carry2):
                        wb_buf[r, :] = wb_buf[r, :] * dsq_buf[r, :]
                        return carry2

                    lax.fori_loop(0, WB, _mul, 0)
                    pltpu.sync_copy(
                        wb_buf,
                        zs_out[layer].at[pl.ds(g * NPAD + r0, WB)])
                return carry

            lax.fori_loop(0, ROWS_PER_TILE // WB, _wb, 0)
            plsc.subcore_barrier()


@functools.cache
def _mega_call():
    return pl.kernel(
        _mega_body,
        out_type=(
            jax.ShapeDtypeStruct((NPAD, D), jnp.float32),
            jax.ShapeDtypeStruct((NPAD, D), jnp.float32),
            jax.ShapeDtypeStruct((NPAD, D), jnp.float32),
            jax.ShapeDtypeStruct((NGRP * NPAD, 16), jnp.float32),
            jax.ShapeDtypeStruct((NGRP * NPAD, 16), jnp.float32),
        ),
        mesh=_mesh(),
        compiler_params=pltpu.CompilerParams(use_tc_tiling_on_sc=False),
        scratch_types=[
            pltpu.VMEM_SHARED((NPAD + 8, 16), jnp.float32),
            pltpu.VMEM((4, QE), jnp.int32),
            pltpu.VMEM((4, QE), jnp.int32),
            pltpu.VMEM((256, 16), jnp.float32),
            pltpu.VMEM((256, 16), jnp.float32),
            pltpu.VMEM((256, 16), jnp.float32),
            pltpu.VMEM((256, 16), jnp.float32),
            pltpu.VMEM((ZB, 16), jnp.float32),
            pltpu.VMEM((WB, 16), jnp.float32),
            pltpu.VMEM((WB, 16), jnp.float32),
            pltpu.SemaphoreType.DMA,
            pltpu.SemaphoreType.DMA,
            pltpu.SemaphoreType.DMA,
            pltpu.SemaphoreType.DMA,
        ],
    )


_BLK = 6272
_GRID = NPAD // _BLK


def _scale_body(a_ref, b_ref, o_ref):
    o_ref[:, :] = a_ref[:, :] * b_ref[:, :]


_scale_call = pl.pallas_call(
    _scale_body,
    grid=(_GRID,),
    in_specs=[pl.BlockSpec((_BLK, D), lambda i: (i, 0)),
              pl.BlockSpec((_BLK, D), lambda i: (i, 0))],
    out_specs=pl.BlockSpec((_BLK, D), lambda i: (i, 0)),
    out_shape=jax.ShapeDtypeStruct((NPAD, D), jnp.float32),
)


def _dsq_body(d_ref, o_ref):
    d16 = d_ref[:, 0:16]
    o_ref[:, :] = d16 * d16


_dsq_call = pl.pallas_call(
    _dsq_body,
    grid=(_GRID,),
    in_specs=[pl.BlockSpec((_BLK, D), lambda j: (j, 0))],
    out_specs=pl.BlockSpec((_BLK, 16), lambda j: (j, 0)),
    out_shape=jax.ShapeDtypeStruct((NPAD, 16), jnp.float32),
)


def _final_body(x_ref, d_ref, s1_ref, s2_ref, s3_ref, o_ref):
    acc = s1_ref[:, :] + s2_ref[:, :] + s3_ref[:, :]
    o_ref[:, :] = (x_ref[:, :] + d_ref[:, :] * acc) * (1.0 / 16.0)


_final_call = pl.pallas_call(
    _final_body,
    grid=(_GRID,),
    in_specs=[pl.BlockSpec((_BLK, D), lambda i: (i, 0)) for _ in range(5)],
    out_specs=pl.BlockSpec((_BLK, D), lambda i: (i, 0)),
    out_shape=jax.ShapeDtypeStruct((NPAD, D), jnp.float32),
)


def kernel(edge_index, user_int, item_int, user_geo, item_geo):
    src = edge_index[0].astype(jnp.int32)
    dst = edge_index[1].astype(jnp.int32)

    x_int = jnp.concatenate([user_int, item_int], axis=0)
    x_geo = jnp.concatenate([user_geo, item_geo], axis=0)
    x0 = jnp.concatenate([x_int, x_geo], axis=1)
    x0p = jnp.pad(x0, ((0, NPAD - N), (0, 0)))

    pad_e = EPAD - E
    src_p = jnp.concatenate([src, jnp.zeros((pad_e,), jnp.int32)])
    dst_p = jnp.concatenate([dst, jnp.full((pad_e,), NPAD, jnp.int32)])
    src3d = src_p.reshape(EPAD // QE, QE)
    dst3d = dst_p.reshape(EPAD // QE, QE)
    dst2d = dst_p.reshape(EPAD // 128, 128)

    degp = _deg_call()(dst2d)
    deg = degp[0] + degp[1]
    dis = jnp.where(deg > 0, 1.0 / jnp.sqrt(jnp.where(deg > 0, deg, 1.0)),
                    0.0)
    disb = jnp.broadcast_to(dis[:, None], (NPAD, D))

    zfull = _scale_call(x0p, disb)
    z0 = zfull.reshape(NGRP * NPAD, 16)
    dsq16 = _dsq_call(disb)
    s1, s2, s3, _z1, _z2 = _mega_call()(z0, src3d, dst3d, dsq16)

    out = _final_call(x0p, disb, s1, s2, s3)

    return (out[:USER_NUM, :32], out[USER_NUM:N, :32],
            out[:USER_NUM, 32:], out[USER_NUM:N, 32:])
